# baseline (speedup 1.0000x reference)
import jax
import jax.numpy as jnp
from jax.experimental import pallas as pl
from jax.experimental.pallas import tpu as pltpu

NH = 8
HD = 16
CT = -0.2260138304488262

_LEVELS = [0.01, 0.1, 0.11, 0.2, 0.21, 0.3, 0.31, 0.4, 0.41,
           0.5, 0.51, 0.6, 0.61, 0.7, 0.71, 0.8, 0.81, 0.9, 0.91]


def _interp_consts(n):
    import numpy as np
    out = []
    for lv in _LEVELS:
        pos = np.float32(lv) * np.float32(n - 1)
        a = int(np.floor(pos))
        hw = np.float32(pos) - np.float32(a)
        lw = np.float32(1.0) - hw
        out.append((a, float(lw), float(hw)))
    return out


def _roll_l(a, j):
    return pltpu.roll(a, a.shape[1] - j, 1)


def _roll_r(a, j):
    return pltpu.roll(a, j, 1)


def _mask_kernel(xp_ref, xtp_ref, out_ref):
    xp = xp_ref[0]
    xtp = xtp_ref[0]
    n = xp.shape[0]
    d = jnp.abs(xp[:, 0:1] - xtp[0:1, :])
    for s in range(1, 12):
        d = d + jnp.abs(xp[:, s:s + 1] - xtp[s:s + 1, :])
    lane = jax.lax.broadcasted_iota(jnp.int32, (n, n), 1)
    lf = jax.lax.broadcasted_iota(jnp.int32, (4 * n, 128), 1)
    gf = jax.lax.broadcasted_iota(jnp.int32, (4 * n, 128), 0) & 3
    one = jnp.float32(1)
    neg = jnp.float32(-1)

    def fullbit(kk):
        if kk < 128:
            return (lf & kk) != 0
        return (gf & (kk // 128)) != 0

    def intra_stages(tf, k):
        j = min(k // 2, 64)
        while j >= 1:
            bit = (lf & j) != 0
            partner = jnp.take_along_axis(tf, lf ^ j, axis=1)
            lo = jnp.minimum(tf, partner)
            hi = jnp.maximum(tf, partner)
            tf = jnp.where(bit, hi, lo)
            j //= 2
        return tf

    def cross_stage(t, j):
        bit = (lane & j) != 0
        partner = jnp.where(bit, _roll_r(t, j), _roll_l(t, j))
        lo = jnp.minimum(t, partner)
        hi = jnp.maximum(t, partner)
        return jnp.where(bit, hi, lo)

    tf = jnp.reshape(d, (4 * n, 128))
    k = 2
    while k <= 128:
        par = fullbit(2) if k == 2 else (fullbit(k) != fullbit(k // 2))
        tf = tf * jnp.where(par, neg, one)
        tf = intra_stages(tf, k)
        k *= 2
    tf = tf * jnp.where(fullbit(256) != fullbit(128), neg, one)
    t = jnp.reshape(tf, (n, n))
    t = cross_stage(t, 128)
    tf = intra_stages(jnp.reshape(t, (4 * n, 128)), 128)
    tf = tf * jnp.where(fullbit(256), neg, one)
    t = jnp.reshape(tf, (n, n))
    t = cross_stage(t, 256)
    t = cross_stage(t, 128)
    tf = intra_stages(jnp.reshape(t, (4 * n, 128)), 128)
    srt = jnp.reshape(tf, (n, n))
    consts = _interp_consts(n)

    def thr(i):
        a, lw, hw = consts[i]
        return srt[:, a:a + 1] * jnp.float32(lw) + srt[:, a + 1:a + 2] * jnp.float32(hw)

    m = d <= thr(0)
    for i in range(9):
        m = m | ((d >= thr(1 + 2 * i)) & (d <= thr(2 + 2 * i)))
    xmean = jnp.sum(xtp, axis=0, keepdims=True) / 12.0
    m = m | (xmean <= CT)
    out_ref[0] = m.astype(jnp.bfloat16)


def _attn_kernel(q_ref, k_ref, v_ref, m_ref, wq_ref, bq_ref, wk_ref, bk_ref,
                 wv_ref, bv_ref, wo_ref, bo_ref, out_ref):
    f32 = jnp.float32
    bf16 = jnp.bfloat16
    maskb = m_ref[0]
    qb = (jnp.dot(q_ref[0, 0], wq_ref[...], preferred_element_type=f32)
          + bq_ref[...]).astype(bf16)
    kb = (jnp.dot(k_ref[0, 0], wk_ref[...], preferred_element_type=f32)
          + bk_ref[...]).astype(bf16)
    vb = (jnp.dot(v_ref[0, 0], wv_ref[...], preferred_element_type=f32)
          + bv_ref[...]).astype(bf16)
    cols = []
    for h in range(NH):
        sl = slice(h * HD, (h + 1) * HD)
        sc = jax.lax.dot_general(qb[:, sl], kb[:, sl], (((1,), (1,)), ((), ())),
                                 preferred_element_type=f32)
        p = jnp.exp(sc.astype(bf16)) * maskb
        denom = jnp.sum(p, axis=1, keepdims=True, dtype=f32)
        cols.append(jnp.dot(p, vb[:, sl], preferred_element_type=f32) / denom)
    o = jnp.concatenate(cols, axis=1)
    out_ref[0, 0] = jnp.dot(o, wo_ref[...], preferred_element_type=f32) + bo_ref[...]


def kernel(query, key, value, x, distance_matrix, Wq, bq, Wk, bk, Wv, bv, Wo, bo, SCALER):
    del distance_matrix, SCALER
    B, S, N, D = query.shape
    xp = jnp.pad(x, ((0, 0), (0, 0), (0, 128 - x.shape[-1])))
    xtp = jnp.transpose(xp, (0, 2, 1))

    maskf = pl.pallas_call(
        _mask_kernel,
        grid=(B,),
        in_specs=[
            pl.BlockSpec((1, N, 128), lambda b: (b, 0, 0)),
            pl.BlockSpec((1, 128, N), lambda b: (b, 0, 0)),
        ],
        out_specs=pl.BlockSpec((1, N, N), lambda b: (b, 0, 0)),
        out_shape=jax.ShapeDtypeStruct((B, N, N), jnp.bfloat16),
    )(xp, xtp)

    wspec = pl.BlockSpec((D, D), lambda b, s: (0, 0))
    bspec = pl.BlockSpec((1, D), lambda b, s: (0, 0))
    qkv_spec = pl.BlockSpec((1, 1, N, D), lambda b, s: (b, s, 0, 0))
    out = pl.pallas_call(
        _attn_kernel,
        grid=(B, S),
        in_specs=[
            qkv_spec, qkv_spec, qkv_spec,
            pl.BlockSpec((1, N, N), lambda b, s: (b, 0, 0)),
            wspec, bspec, wspec, bspec, wspec, bspec, wspec, bspec,
        ],
        out_specs=qkv_spec,
        out_shape=jax.ShapeDtypeStruct((B, S, N, D), jnp.float32),
    )(query, key, value, maskf,
      Wq.T * 0.25, bq.reshape(1, D) * 0.25, Wk.T, bk.reshape(1, D),
      Wv.T, bv.reshape(1, D), Wo.T, bo.reshape(1, D))
    return out


# speedup vs baseline: 1.0384x; 1.0384x over previous
import jax
import jax.numpy as jnp
from jax.experimental import pallas as pl

NH = 8
HD = 16
CT = -0.2260138304488262

_LEVELS = [0.01, 0.1, 0.11, 0.2, 0.21, 0.3, 0.31, 0.4, 0.41,
           0.5, 0.51, 0.6, 0.61, 0.7, 0.71, 0.8, 0.81, 0.9, 0.91]


def _interp_consts(n):
    import numpy as np
    out = []
    for lv in _LEVELS:
        pos = np.float32(lv) * np.float32(n - 1)
        a = int(np.floor(pos))
        hw = np.float32(pos) - np.float32(a)
        lw = np.float32(1.0) - hw
        out.append((a, float(lw), float(hw)))
    return out


def _roll_l(a, j):
    return jnp.concatenate([a[:, j:], a[:, :j]], axis=1)


def _roll_r(a, j):
    return jnp.concatenate([a[:, -j:], a[:, :-j]], axis=1)


def _mask_kernel(xp_ref, xtp_ref, out_ref):
    xp = xp_ref[0]
    xtp = xtp_ref[0]
    n = xp.shape[0]
    d = jnp.abs(xp[:, 0:1] - xtp[0:1, :])
    for s in range(1, 12):
        d = d + jnp.abs(xp[:, s:s + 1] - xtp[s:s + 1, :])
    lane = jax.lax.broadcasted_iota(jnp.int32, (n, n), 1)
    srt = d
    k = 2
    while k <= n:
        j = k // 2
        while j >= 1:
            bit = (lane & j) != 0
            partner = jnp.where(bit, _roll_r(srt, j), _roll_l(srt, j))
            lo = jnp.minimum(srt, partner)
            hi = jnp.maximum(srt, partner)
            desc = (lane & k) != 0
            srt = jnp.where(bit != desc, hi, lo)
            j //= 2
        k *= 2
    consts = _interp_consts(n)

    def thr(i):
        a, lw, hw = consts[i]
        return srt[:, a:a + 1] * jnp.float32(lw) + srt[:, a + 1:a + 2] * jnp.float32(hw)

    m = d <= thr(0)
    for i in range(9):
        m = m | ((d >= thr(1 + 2 * i)) & (d <= thr(2 + 2 * i)))
    xmean = jnp.sum(xtp, axis=0, keepdims=True) / 12.0
    m = m | (xmean <= CT)
    out_ref[0] = m.astype(jnp.float32)


def _attn_kernel(q_ref, k_ref, v_ref, m_ref, wq_ref, bq_ref, wk_ref, bk_ref,
                 wv_ref, bv_ref, wo_ref, bo_ref, out_ref):
    f32 = jnp.float32
    bf16 = jnp.bfloat16
    maskf = m_ref[0]
    qb = (jnp.dot(q_ref[0, 0], wq_ref[...], preferred_element_type=f32)
          + bq_ref[...]).astype(bf16)
    kb = (jnp.dot(k_ref[0, 0], wk_ref[...], preferred_element_type=f32)
          + bk_ref[...]).astype(bf16)
    vb = (jnp.dot(v_ref[0, 0], wv_ref[...], preferred_element_type=f32)
          + bv_ref[...]).astype(bf16)
    cols = []
    for h in range(NH):
        sl = slice(h * HD, (h + 1) * HD)
        sc = jax.lax.dot_general(qb[:, sl], kb[:, sl], (((1,), (1,)), ((), ())),
                                 preferred_element_type=f32)
        p = jnp.exp(sc) * maskf
        denom = jnp.sum(p, axis=1, keepdims=True)
        pb = p.astype(bf16)
        cols.append(jnp.dot(pb, vb[:, sl], preferred_element_type=f32) / denom)
    o = jnp.concatenate(cols, axis=1)
    out_ref[0, 0] = jnp.dot(o, wo_ref[...], preferred_element_type=f32) + bo_ref[...]


def kernel(query, key, value, x, distance_matrix, Wq, bq, Wk, bk, Wv, bv, Wo, bo, SCALER):
    del distance_matrix, SCALER
    B, S, N, D = query.shape
    xp = jnp.pad(x, ((0, 0), (0, 0), (0, 128 - x.shape[-1])))
    xtp = jnp.transpose(xp, (0, 2, 1))

    maskf = pl.pallas_call(
        _mask_kernel,
        grid=(B,),
        in_specs=[
            pl.BlockSpec((1, N, 128), lambda b: (b, 0, 0)),
            pl.BlockSpec((1, 128, N), lambda b: (b, 0, 0)),
        ],
        out_specs=pl.BlockSpec((1, N, N), lambda b: (b, 0, 0)),
        out_shape=jax.ShapeDtypeStruct((B, N, N), jnp.float32),
    )(xp, xtp)

    wspec = pl.BlockSpec((D, D), lambda b, s: (0, 0))
    bspec = pl.BlockSpec((1, D), lambda b, s: (0, 0))
    qkv_spec = pl.BlockSpec((1, 1, N, D), lambda b, s: (b, s, 0, 0))
    out = pl.pallas_call(
        _attn_kernel,
        grid=(B, S),
        in_specs=[
            qkv_spec, qkv_spec, qkv_spec,
            pl.BlockSpec((1, N, N), lambda b, s: (b, 0, 0)),
            wspec, bspec, wspec, bspec, wspec, bspec, wspec, bspec,
        ],
        out_specs=qkv_spec,
        out_shape=jax.ShapeDtypeStruct((B, S, N, D), jnp.float32),
    )(query, key, value, maskf,
      Wq.T * 0.25, bq.reshape(1, D) * 0.25, Wk.T, bk.reshape(1, D),
      Wv.T, bv.reshape(1, D), Wo.T, bo.reshape(1, D))
    return out
